# Initial kernel scaffold; baseline (speedup 1.0000x reference)
import jax
import jax.numpy as jnp
from jax.experimental import pallas as pl


def kernel(x, edge_index, batch, W1, b1, W2, b2):
    raise NotImplementedError("write your pallas kernel here")


# speedup vs baseline: 26.8732x; 26.8732x over previous
import functools

import jax
import jax.numpy as jnp
from jax import lax
from jax.experimental import pallas as pl
from jax.experimental.pallas import tpu as pltpu
from jax.experimental.pallas import tpu_sc as plsc

N = 10000
NP = 10240
F = 128
E = 320000
NC = 2
NS = 16
NW = NC * NS
C = 128
NCH = 80
EP = NW * NCH * C
ROWS_PER_TILE = NP // NS


def _sc_mesh():
    return plsc.VectorSubcoreMesh(core_axis_name="c", subcore_axis_name="s",
                                  num_cores=NC, num_subcores=NS)


def _deg_body(dst_hbm, deg_out, dst_v, buf, sem, deg_sh):
    cid = lax.axis_index("c")
    sid = lax.axis_index("s")
    wid = cid * NS + sid
    H = NCH // 2

    zeros = jnp.zeros((16,), jnp.float32)
    ones = jnp.ones((16,), jnp.float32)

    def zero_row(r, carry):
        for j in range(F // 16):
            buf[r, pl.ds(j * 16, 16)] = zeros
        return carry

    lax.fori_loop(0, C, zero_row, 0)
    base = sid * ROWS_PER_TILE
    for i in range(ROWS_PER_TILE // C):
        pltpu.sync_copy(buf, deg_sh.at[pl.ds(base + i * C, C)])

    def ones_row(r, carry):
        for j in range(F // 16):
            buf[r, pl.ds(j * 16, 16)] = ones
        return carry

    lax.fori_loop(0, C, ones_row, 0)
    plsc.subcore_barrier()

    for p in range(2):
        pltpu.async_copy(dst_hbm.at[wid, pl.ds(p * H, H)], dst_v, sem).wait()
        for k in range(H):
            pltpu.sync_copy(buf, deg_sh.at[dst_v.at[k]], add=True)
    plsc.subcore_barrier()

    for i in range(ROWS_PER_TILE // C):
        sl = pl.ds(base + i * C, C)
        pltpu.sync_copy(deg_sh.at[sl], buf)
        pltpu.sync_copy(buf, deg_out.at[cid, sl])


def _deg_call(dst3):
    return pl.kernel(
        _deg_body,
        out_type=jax.ShapeDtypeStruct((NC, NP, F), jnp.float32),
        mesh=_sc_mesh(),
        scratch_types=[
            pltpu.VMEM((NCH // 2, C), jnp.int32),
            pltpu.VMEM((C, F), jnp.float32),
            pltpu.SemaphoreType.DMA,
            pltpu.VMEM_SHARED((NP, F), jnp.float32),
        ],
    )(dst3)


def _agg_body(g_hbm, src_hbm, dst_hbm, acc_out, src_v, dst_v, buf0, buf1,
              sem_i, sem0, sem1, acc_sh):
    cid = lax.axis_index("c")
    sid = lax.axis_index("s")
    wid = cid * NS + sid
    H = NCH // 2

    zeros = jnp.zeros((16,), jnp.float32)

    def zero_row(r, carry):
        for j in range(F // 16):
            buf0[r, pl.ds(j * 16, 16)] = zeros
        return carry

    lax.fori_loop(0, C, zero_row, 0)
    base = sid * ROWS_PER_TILE
    for i in range(ROWS_PER_TILE // C):
        pltpu.sync_copy(buf0, acc_sh.at[pl.ds(base + i * C, C)])
    plsc.subcore_barrier()

    for p in range(2):
        pltpu.async_copy(src_hbm.at[wid, pl.ds(p * H, H)], src_v, sem_i).wait()
        pltpu.async_copy(dst_hbm.at[wid, pl.ds(p * H, H)], dst_v, sem_i).wait()
        d0 = pltpu.async_copy(g_hbm.at[src_v.at[0]], buf0, sem0)
        d1 = pltpu.async_copy(g_hbm.at[src_v.at[1]], buf1, sem1)
        for k in range(H):
            buf, sem, d = (buf0, sem0, d0) if k % 2 == 0 else (buf1, sem1, d1)
            d.wait()
            pltpu.sync_copy(buf, acc_sh.at[dst_v.at[k]], add=True)
            if k + 2 < H:
                d_new = pltpu.async_copy(g_hbm.at[src_v.at[k + 2]], buf, sem)
                if k % 2 == 0:
                    d0 = d_new
                else:
                    d1 = d_new
    plsc.subcore_barrier()

    for i in range(ROWS_PER_TILE // C):
        sl = pl.ds(base + i * C, C)
        pltpu.sync_copy(acc_sh.at[sl], buf0)
        pltpu.sync_copy(buf0, acc_out.at[cid, sl])


def _agg_call(g, src3, dst3):
    return pl.kernel(
        _agg_body,
        out_type=jax.ShapeDtypeStruct((NC, NP, F), jnp.float32),
        mesh=_sc_mesh(),
        scratch_types=[
            pltpu.VMEM((NCH // 2, C), jnp.int32),
            pltpu.VMEM((NCH // 2, C), jnp.int32),
            pltpu.VMEM((C, F), jnp.float32),
            pltpu.VMEM((C, F), jnp.float32),
            pltpu.SemaphoreType.DMA,
            pltpu.SemaphoreType.DMA,
            pltpu.SemaphoreType.DMA,
            pltpu.VMEM_SHARED((NP, F), jnp.float32),
        ],
    )(g, src3, dst3)


_BLK = 640
_GRID = NP // _BLK

def _k1_body(x_ref, w_ref, deg_ref, g_ref, dis_ref):
    dis = lax.rsqrt(deg_ref[0] + deg_ref[1] + 1.0)
    h = jnp.dot(x_ref[...], w_ref[...], preferred_element_type=jnp.float32)
    dis_ref[...] = dis
    g_ref[...] = h * dis


def _k1_call(x_p, W1, degs):
    return pl.pallas_call(
        _k1_body,
        out_shape=(jax.ShapeDtypeStruct((NP, F), jnp.float32),
                   jax.ShapeDtypeStruct((NP, F), jnp.float32)),
        grid=(_GRID,),
        in_specs=[
            pl.BlockSpec((_BLK, F), lambda i: (i, 0)),
            pl.BlockSpec((F, F), lambda i: (0, 0)),
            pl.BlockSpec((NC, _BLK, F), lambda i: (0, i, 0)),
        ],
        out_specs=(pl.BlockSpec((_BLK, F), lambda i: (i, 0)),
                   pl.BlockSpec((_BLK, F), lambda i: (i, 0))),
    )(x_p, W1, degs)


def _k3_body(acc_ref, g1_ref, dis_ref, b1_ref, w2_ref, g2_ref):
    dis = dis_ref[...]
    s = acc_ref[0] + acc_ref[1] + g1_ref[...]
    o1 = jnp.maximum(s * dis + b1_ref[...], 0.0)
    h2 = jnp.dot(o1, w2_ref[...], preferred_element_type=jnp.float32)
    g2_ref[...] = h2 * dis


def _k3_call(acc1, g1, dis2d, b1_2d, W2):
    return pl.pallas_call(
        _k3_body,
        out_shape=jax.ShapeDtypeStruct((NP, F), jnp.float32),
        grid=(_GRID,),
        in_specs=[
            pl.BlockSpec((NC, _BLK, F), lambda i: (0, i, 0)),
            pl.BlockSpec((_BLK, F), lambda i: (i, 0)),
            pl.BlockSpec((_BLK, F), lambda i: (i, 0)),
            pl.BlockSpec((1, F), lambda i: (0, 0)),
            pl.BlockSpec((F, F), lambda i: (0, 0)),
        ],
        out_specs=pl.BlockSpec((_BLK, F), lambda i: (i, 0)),
    )(acc1, g1, dis2d, b1_2d, W2)


def _k5_body(acc_ref, g2_ref, dis_ref, b2_ref, out_ref):
    z = (acc_ref[0] + acc_ref[1] + g2_ref[...]) * dis_ref[...] + b2_ref[...]
    out_ref[...] = 1.0 / (1.0 + jnp.exp(-z))


def _k5_call(acc2, g2, dis2d, b2_2d):
    return pl.pallas_call(
        _k5_body,
        out_shape=jax.ShapeDtypeStruct((NP, F), jnp.float32),
        grid=(_GRID,),
        in_specs=[
            pl.BlockSpec((NC, _BLK, F), lambda i: (0, i, 0)),
            pl.BlockSpec((_BLK, F), lambda i: (i, 0)),
            pl.BlockSpec((_BLK, F), lambda i: (i, 0)),
            pl.BlockSpec((1, F), lambda i: (0, 0)),
        ],
        out_specs=pl.BlockSpec((_BLK, F), lambda i: (i, 0)),
    )(acc2, g2, dis2d, b2_2d)


def kernel(x, edge_index, batch, W1, b1, W2, b2):
    src = edge_index[0].astype(jnp.int32)
    dst = edge_index[1].astype(jnp.int32)
    pad = N + (jnp.arange(EP - E, dtype=jnp.int32) % (NP - N))
    src3 = jnp.concatenate([src, pad]).reshape(NW, NCH, C)
    dst3 = jnp.concatenate([dst, pad]).reshape(NW, NCH, C)
    x_p = jnp.zeros((NP, F), jnp.float32).at[:N].set(x)

    degs = _deg_call(dst3)
    g1, dis2d = _k1_call(x_p, W1, degs)
    acc1 = _agg_call(g1, src3, dst3)
    g2 = _k3_call(acc1, g1, dis2d, b1.reshape(1, F), W2)
    acc2 = _agg_call(g2, src3, dst3)
    out = _k5_call(acc2, g2, dis2d, b2.reshape(1, F))
    return out[:N]
